# baseline (speedup 1.0000x reference)
import functools

import jax
import jax.numpy as jnp
from jax import lax
from jax.experimental import pallas as pl
from jax.experimental.pallas import tpu as pltpu
from jax.experimental.pallas import tpu_sc as plsc

EMB = 128
Q = 32
FAN = EMB + 2 * Q
NC = 2
NS = 16
NW = NC * NS
CH = 128
LANES = 8
TABLE_BLK = 2000
NBUF = 4


def _proj_body(emb_ref, fw_ref, pw_ref, pb_ref, aw_ref, ab_ref, fb_ref,
               p_ref, vp_ref, va_ref):
    fw = fw_ref[...]
    w1 = fw[:, :EMB]
    w2 = fw[:, EMB:EMB + Q]
    w3 = fw[:, EMB + Q:]
    dn = (((1,), (1,)), ((), ()))
    c = (jnp.sum(w2 * pb_ref[...], axis=1) + jnp.sum(w3 * ab_ref[...], axis=1)
         + fb_ref[0, :])
    p_ref[...] = lax.dot_general(emb_ref[...], w1, dn,
                                 preferred_element_type=jnp.float32) + c[None, :]
    vp_ref[...] = jnp.sum(w2 * pw_ref[...], axis=1)[None, :]
    va_ref[...] = jnp.sum(w3 * aw_ref[...], axis=1)[None, :]


def _project_table(service_emb, final_W, pw, pb, aw, ab, fb):
    rows = service_emb.shape[0]
    grid = rows // TABLE_BLK
    zero = lambda i: (0, 0)
    return pl.pallas_call(
        _proj_body,
        grid=(grid,),
        in_specs=[
            pl.BlockSpec((TABLE_BLK, EMB), lambda i: (i, 0)),
            pl.BlockSpec((EMB, FAN), zero),
            pl.BlockSpec((1, Q), zero),
            pl.BlockSpec((1, Q), zero),
            pl.BlockSpec((1, Q), zero),
            pl.BlockSpec((1, Q), zero),
            pl.BlockSpec((1, EMB), zero),
        ],
        out_specs=[
            pl.BlockSpec((TABLE_BLK, EMB), lambda i: (i, 0)),
            pl.BlockSpec((1, EMB), zero),
            pl.BlockSpec((1, EMB), zero),
        ],
        out_shape=[
            jax.ShapeDtypeStruct((rows, EMB), jnp.float32),
            jax.ShapeDtypeStruct((1, EMB), jnp.float32),
            jax.ShapeDtypeStruct((1, EMB), jnp.float32),
        ],
    )(service_emb, final_W, pw, pb, aw, ab, fb)


@functools.lru_cache(maxsize=4)
def _make_sc_gather(bn, rows):
    assert bn % (NW * CH) == 0
    tok_per_w = bn // NW
    steps = tok_per_w // CH
    assert steps % NBUF == 0 and steps >= 2 * NBUF
    mesh = plsc.VectorSubcoreMesh(core_axis_name="c", subcore_axis_name="s")

    @functools.partial(
        pl.kernel,
        mesh=mesh,
        out_type=jax.ShapeDtypeStruct((bn, EMB), jnp.float32),
        scratch_types=[
            pltpu.VMEM((tok_per_w,), jnp.int32),
            pltpu.VMEM((tok_per_w,), jnp.float32),
            pltpu.VMEM((tok_per_w,), jnp.float32),
            pltpu.VMEM((NBUF, CH, EMB), jnp.float32),
            pltpu.VMEM((EMB,), jnp.float32),
            pltpu.VMEM((EMB,), jnp.float32),
            pltpu.SemaphoreType.DMA,
            pltpu.SemaphoreType.DMA,
            pltpu.SemaphoreType.DMA,
        ],
    )
    def sc_gather(p_hbm, idx_hbm, ph_hbm, am_hbm, vp_hbm, va_hbm, out_hbm,
                  idx_v, ph_v, am_v, rows_v, vp_v, va_v, gsem, wsem, ssem):
        wid = lax.axis_index("s") * NC + lax.axis_index("c")
        tok0 = wid * tok_per_w
        stages = [
            (vp_hbm.at[0], vp_v), (va_hbm.at[0], va_v),
            (idx_hbm.at[pl.ds(tok0, tok_per_w)], idx_v),
            (ph_hbm.at[pl.ds(tok0, tok_per_w)], ph_v),
            (am_hbm.at[pl.ds(tok0, tok_per_w)], am_v),
        ]
        for src, dst in stages:
            pltpu.async_copy(src, dst, ssem)
        for src, dst in stages:
            pltpu.make_async_copy(src, dst, ssem).wait()
        vps = [vp_v[pl.ds(k * 16, 16)] for k in range(LANES)]
        vas = [va_v[pl.ds(k * 16, 16)] for k in range(LANES)]

        def start_gather(s, b):
            pltpu.async_copy(p_hbm.at[idx_v.at[pl.ds(s * CH, CH)]], rows_v.at[b], gsem)

        def wait_gather(b):
            pltpu.make_async_copy(p_hbm.at[pl.ds(0, CH)], rows_v.at[b], gsem).wait()

        def start_writeback(s, b):
            pltpu.async_copy(rows_v.at[b], out_hbm.at[pl.ds(tok0 + s * CH, CH)], wsem)

        def wait_writeback(b):
            pltpu.make_async_copy(rows_v.at[b], out_hbm.at[pl.ds(tok0, CH)], wsem).wait()

        def compute(s, b):
            def grp(g, inner):
                ph16 = ph_v[pl.ds(s * CH + g * 16, 16)]
                am16 = am_v[pl.ds(s * CH + g * 16, 16)]
                for t in range(16):
                    ph = jnp.full((16,), ph16[t], dtype=jnp.float32)
                    am = jnp.full((16,), am16[t], dtype=jnp.float32)
                    j = g * 16 + t
                    for k in range(LANES):
                        sl = pl.ds(k * 16, 16)
                        rows_v[b, j, sl] = rows_v[b, j, sl] + ph * vps[k] + am * vas[k]
                return inner

            lax.fori_loop(0, CH // 16, grp, 0)

        start_gather(0, 0)
        start_gather(1, 1)

        def block(q, carry):
            for r in range(NBUF):
                s = q * NBUF + r
                b = r
                bg = (r + 2) % NBUF

                if r < 2:
                    cond_wb = q >= 1
                else:
                    cond_wb = True
                if cond_wb is True:
                    wait_writeback(bg)
                else:
                    @pl.when(q >= 1)
                    def _(bg=bg):
                        wait_writeback(bg)

                if r < NBUF - 2:
                    start_gather(s + 2, bg)
                else:
                    @pl.when(s + 2 < steps)
                    def _(s=s, bg=bg):
                        start_gather(s + 2, bg)

                wait_gather(b)
                compute(s, b)
                start_writeback(s, b)
            return carry

        lax.fori_loop(0, steps // NBUF, block, 0)
        wait_writeback(NBUF - 2)
        wait_writeback(NBUF - 1)

    return sc_gather


def kernel(service_indices, phases, amplitudes, service_emb,
           phase_W, phase_b, amp_W, amp_b, final_W, final_b):
    bn = service_indices.shape[0]
    idx = service_indices.astype(jnp.int32)
    ph = phases.reshape(-1)
    am = amplitudes.reshape(-1)
    pw = phase_W.reshape(1, Q)
    aw = amp_W.reshape(1, Q)
    pb = phase_b.reshape(1, Q)
    ab = amp_b.reshape(1, Q)
    fb = final_b.reshape(1, EMB)
    P, vp, va = _project_table(service_emb, final_W, pw, pb, aw, ab, fb)
    sc = _make_sc_gather(bn, service_emb.shape[0])
    return sc(P, idx, ph, am, vp, va)


# speedup vs baseline: 1.0723x; 1.0723x over previous
import functools

import jax
import jax.numpy as jnp
from jax import lax
from jax.experimental import pallas as pl
from jax.experimental.pallas import tpu as pltpu
from jax.experimental.pallas import tpu_sc as plsc

EMB = 128
Q = 32
FAN = EMB + 2 * Q
NC = 2
NS = 16
NW = NC * NS
CH = 128
LANES = 8
TABLE_BLK = 4000
NBUF = 4


def _proj_body(emb_ref, fw_ref, pw_ref, pb_ref, aw_ref, ab_ref, fb_ref,
               p_ref, vp_ref, va_ref):
    fw = fw_ref[...]
    w1 = fw[:, :EMB]
    w2 = fw[:, EMB:EMB + Q]
    w3 = fw[:, EMB + Q:]
    dn = (((1,), (1,)), ((), ()))
    c = (jnp.sum(w2 * pb_ref[...], axis=1) + jnp.sum(w3 * ab_ref[...], axis=1)
         + fb_ref[0, :])
    p_ref[...] = lax.dot_general(emb_ref[...], w1, dn,
                                 preferred_element_type=jnp.float32) + c[None, :]
    vp_ref[...] = jnp.sum(w2 * pw_ref[...], axis=1)[None, :]
    va_ref[...] = jnp.sum(w3 * aw_ref[...], axis=1)[None, :]


def _project_table(service_emb, final_W, pw, pb, aw, ab, fb):
    rows = service_emb.shape[0]
    grid = rows // TABLE_BLK
    zero = lambda i: (0, 0)
    return pl.pallas_call(
        _proj_body,
        grid=(grid,),
        in_specs=[
            pl.BlockSpec((TABLE_BLK, EMB), lambda i: (i, 0)),
            pl.BlockSpec((EMB, FAN), zero),
            pl.BlockSpec((1, Q), zero),
            pl.BlockSpec((1, Q), zero),
            pl.BlockSpec((1, Q), zero),
            pl.BlockSpec((1, Q), zero),
            pl.BlockSpec((1, EMB), zero),
        ],
        out_specs=[
            pl.BlockSpec((TABLE_BLK, EMB), lambda i: (i, 0)),
            pl.BlockSpec((1, EMB), zero),
            pl.BlockSpec((1, EMB), zero),
        ],
        out_shape=[
            jax.ShapeDtypeStruct((rows, EMB), jnp.float32),
            jax.ShapeDtypeStruct((1, EMB), jnp.float32),
            jax.ShapeDtypeStruct((1, EMB), jnp.float32),
        ],
    )(service_emb, final_W, pw, pb, aw, ab, fb)


@functools.lru_cache(maxsize=4)
def _make_sc_gather(bn, rows):
    assert bn % (NW * CH) == 0
    tok_per_w = bn // NW
    steps = tok_per_w // CH
    assert steps % NBUF == 0 and steps >= 2 * NBUF
    mesh = plsc.VectorSubcoreMesh(core_axis_name="c", subcore_axis_name="s")

    @functools.partial(
        pl.kernel,
        mesh=mesh,
        out_type=jax.ShapeDtypeStruct((bn, EMB), jnp.float32),
        scratch_types=[
            pltpu.VMEM((tok_per_w,), jnp.int32),
            pltpu.VMEM((tok_per_w,), jnp.float32),
            pltpu.VMEM((tok_per_w,), jnp.float32),
            pltpu.VMEM((NBUF, CH, EMB), jnp.float32),
            pltpu.VMEM((EMB,), jnp.float32),
            pltpu.VMEM((EMB,), jnp.float32),
            pltpu.SemaphoreType.DMA,
            pltpu.SemaphoreType.DMA,
            pltpu.SemaphoreType.DMA,
        ],
    )
    def sc_gather(p_hbm, idx_hbm, ph_hbm, am_hbm, vp_hbm, va_hbm, out_hbm,
                  idx_v, ph_v, am_v, rows_v, vp_v, va_v, gsem, wsem, ssem):
        wid = lax.axis_index("s") * NC + lax.axis_index("c")
        tok0 = wid * tok_per_w
        pltpu.sync_copy(idx_hbm.at[pl.ds(tok0, tok_per_w)], idx_v)
        stages = [
            (vp_hbm.at[0], vp_v), (va_hbm.at[0], va_v),
            (ph_hbm.at[pl.ds(tok0, tok_per_w)], ph_v),
            (am_hbm.at[pl.ds(tok0, tok_per_w)], am_v),
        ]
        for src, dst in stages:
            pltpu.async_copy(src, dst, ssem)

        def start_gather(s, b):
            pltpu.async_copy(p_hbm.at[idx_v.at[pl.ds(s * CH, CH)]], rows_v.at[b], gsem)

        def wait_gather(b):
            pltpu.make_async_copy(p_hbm.at[pl.ds(0, CH)], rows_v.at[b], gsem).wait()

        def start_writeback(s, b):
            pltpu.async_copy(rows_v.at[b], out_hbm.at[pl.ds(tok0 + s * CH, CH)], wsem)

        def wait_writeback(b):
            pltpu.make_async_copy(rows_v.at[b], out_hbm.at[pl.ds(tok0, CH)], wsem).wait()

        def compute(s, b):
            def grp(g, inner):
                ph16 = ph_v[pl.ds(s * CH + g * 16, 16)]
                am16 = am_v[pl.ds(s * CH + g * 16, 16)]
                for t in range(16):
                    ph = jnp.full((16,), ph16[t], dtype=jnp.float32)
                    am = jnp.full((16,), am16[t], dtype=jnp.float32)
                    j = g * 16 + t
                    for k in range(LANES):
                        sl = pl.ds(k * 16, 16)
                        rows_v[b, j, sl] = rows_v[b, j, sl] + ph * vps[k] + am * vas[k]
                return inner

            lax.fori_loop(0, CH // 16, grp, 0)

        start_gather(0, 0)
        start_gather(1, 1)
        for src, dst in stages:
            pltpu.make_async_copy(src, dst, ssem).wait()
        vps = [vp_v[pl.ds(k * 16, 16)] for k in range(LANES)]
        vas = [va_v[pl.ds(k * 16, 16)] for k in range(LANES)]

        def block(q, carry):
            for r in range(NBUF):
                s = q * NBUF + r
                b = r
                bg = (r + 2) % NBUF

                if r < 2:
                    cond_wb = q >= 1
                else:
                    cond_wb = True
                if cond_wb is True:
                    wait_writeback(bg)
                else:
                    @pl.when(q >= 1)
                    def _(bg=bg):
                        wait_writeback(bg)

                if r < NBUF - 2:
                    start_gather(s + 2, bg)
                else:
                    @pl.when(s + 2 < steps)
                    def _(s=s, bg=bg):
                        start_gather(s + 2, bg)

                wait_gather(b)
                compute(s, b)
                start_writeback(s, b)
            return carry

        lax.fori_loop(0, steps // NBUF, block, 0)
        wait_writeback(NBUF - 2)
        wait_writeback(NBUF - 1)

    return sc_gather


def kernel(service_indices, phases, amplitudes, service_emb,
           phase_W, phase_b, amp_W, amp_b, final_W, final_b):
    bn = service_indices.shape[0]
    idx = service_indices.astype(jnp.int32)
    ph = phases.reshape(-1)
    am = amplitudes.reshape(-1)
    pw = phase_W.reshape(1, Q)
    aw = amp_W.reshape(1, Q)
    pb = phase_b.reshape(1, Q)
    ab = amp_b.reshape(1, Q)
    fb = final_b.reshape(1, EMB)
    P, vp, va = _project_table(service_emb, final_W, pw, pb, aw, ab, fb)
    sc = _make_sc_gather(bn, service_emb.shape[0])
    return sc(P, idx, ph, am, vp, va)


# speedup vs baseline: 1.0729x; 1.0006x over previous
import functools

import jax
import jax.numpy as jnp
from jax import lax
from jax.experimental import pallas as pl
from jax.experimental.pallas import tpu as pltpu
from jax.experimental.pallas import tpu_sc as plsc

EMB = 128
Q = 32
FAN = EMB + 2 * Q
NC = 2
NS = 16
NW = NC * NS
CH = 128
LANES = 8
TABLE_BLK = 4000
NBUF = 5


def _proj_body(emb_ref, fw_ref, pw_ref, pb_ref, aw_ref, ab_ref, fb_ref,
               p_ref, vp_ref, va_ref):
    fw = fw_ref[...]
    w1 = fw[:, :EMB]
    w2 = fw[:, EMB:EMB + Q]
    w3 = fw[:, EMB + Q:]
    dn = (((1,), (1,)), ((), ()))
    c = (jnp.sum(w2 * pb_ref[...], axis=1) + jnp.sum(w3 * ab_ref[...], axis=1)
         + fb_ref[0, :])
    p_ref[...] = lax.dot_general(emb_ref[...], w1, dn,
                                 preferred_element_type=jnp.float32) + c[None, :]
    vp_ref[...] = jnp.sum(w2 * pw_ref[...], axis=1)[None, :]
    va_ref[...] = jnp.sum(w3 * aw_ref[...], axis=1)[None, :]


def _project_table(service_emb, final_W, pw, pb, aw, ab, fb):
    rows = service_emb.shape[0]
    grid = rows // TABLE_BLK
    zero = lambda i: (0, 0)
    return pl.pallas_call(
        _proj_body,
        grid=(grid,),
        in_specs=[
            pl.BlockSpec((TABLE_BLK, EMB), lambda i: (i, 0)),
            pl.BlockSpec((EMB, FAN), zero),
            pl.BlockSpec((1, Q), zero),
            pl.BlockSpec((1, Q), zero),
            pl.BlockSpec((1, Q), zero),
            pl.BlockSpec((1, Q), zero),
            pl.BlockSpec((1, EMB), zero),
        ],
        out_specs=[
            pl.BlockSpec((TABLE_BLK, EMB), lambda i: (i, 0)),
            pl.BlockSpec((1, EMB), zero),
            pl.BlockSpec((1, EMB), zero),
        ],
        out_shape=[
            jax.ShapeDtypeStruct((rows, EMB), jnp.float32),
            jax.ShapeDtypeStruct((1, EMB), jnp.float32),
            jax.ShapeDtypeStruct((1, EMB), jnp.float32),
        ],
    )(service_emb, final_W, pw, pb, aw, ab, fb)


@functools.lru_cache(maxsize=4)
def _make_sc_gather(bn, rows):
    assert bn % (NW * CH) == 0
    tok_per_w = bn // NW
    steps = tok_per_w // CH
    assert steps % NBUF == 0 and steps >= 2 * NBUF
    mesh = plsc.VectorSubcoreMesh(core_axis_name="c", subcore_axis_name="s")

    @functools.partial(
        pl.kernel,
        mesh=mesh,
        out_type=jax.ShapeDtypeStruct((bn, EMB), jnp.float32),
        scratch_types=[
            pltpu.VMEM((tok_per_w,), jnp.int32),
            pltpu.VMEM((tok_per_w,), jnp.float32),
            pltpu.VMEM((tok_per_w,), jnp.float32),
            pltpu.VMEM((NBUF, CH, EMB), jnp.float32),
            pltpu.VMEM((EMB,), jnp.float32),
            pltpu.VMEM((EMB,), jnp.float32),
            pltpu.SemaphoreType.DMA,
            pltpu.SemaphoreType.DMA,
            pltpu.SemaphoreType.DMA,
        ],
    )
    def sc_gather(p_hbm, idx_hbm, ph_hbm, am_hbm, vp_hbm, va_hbm, out_hbm,
                  idx_v, ph_v, am_v, rows_v, vp_v, va_v, gsem, wsem, ssem):
        wid = lax.axis_index("s") * NC + lax.axis_index("c")
        tok0 = wid * tok_per_w
        pltpu.sync_copy(idx_hbm.at[pl.ds(tok0, tok_per_w)], idx_v)
        stages = [
            (vp_hbm.at[0], vp_v), (va_hbm.at[0], va_v),
            (ph_hbm.at[pl.ds(tok0, tok_per_w)], ph_v),
            (am_hbm.at[pl.ds(tok0, tok_per_w)], am_v),
        ]
        for src, dst in stages:
            pltpu.async_copy(src, dst, ssem)

        def start_gather(s, b):
            pltpu.async_copy(p_hbm.at[idx_v.at[pl.ds(s * CH, CH)]], rows_v.at[b], gsem)

        def wait_gather(b):
            pltpu.make_async_copy(p_hbm.at[pl.ds(0, CH)], rows_v.at[b], gsem).wait()

        def start_writeback(s, b):
            pltpu.async_copy(rows_v.at[b], out_hbm.at[pl.ds(tok0 + s * CH, CH)], wsem)

        def wait_writeback(b):
            pltpu.make_async_copy(rows_v.at[b], out_hbm.at[pl.ds(tok0, CH)], wsem).wait()

        def compute(s, b):
            def grp(g, inner):
                ph16 = ph_v[pl.ds(s * CH + g * 16, 16)]
                am16 = am_v[pl.ds(s * CH + g * 16, 16)]
                for t in range(16):
                    ph = jnp.full((16,), ph16[t], dtype=jnp.float32)
                    am = jnp.full((16,), am16[t], dtype=jnp.float32)
                    j = g * 16 + t
                    for k in range(LANES):
                        sl = pl.ds(k * 16, 16)
                        rows_v[b, j, sl] = rows_v[b, j, sl] + ph * vps[k] + am * vas[k]
                return inner

            lax.fori_loop(0, CH // 16, grp, 0)

        start_gather(0, 0)
        start_gather(1, 1)
        start_gather(2, 2)
        for src, dst in stages:
            pltpu.make_async_copy(src, dst, ssem).wait()
        vps = [vp_v[pl.ds(k * 16, 16)] for k in range(LANES)]
        vas = [va_v[pl.ds(k * 16, 16)] for k in range(LANES)]

        def block(q, carry):
            for r in range(NBUF):
                s = q * NBUF + r
                b = r
                bg = (r + 3) % NBUF

                if r >= 2:
                    wait_writeback(bg)
                else:
                    @pl.when(q >= 1)
                    def _(bg=bg):
                        wait_writeback(bg)

                if r < 2:
                    start_gather(s + 3, bg)
                else:
                    @pl.when(q < steps // NBUF - 1)
                    def _(s=s, bg=bg):
                        start_gather(s + 3, bg)

                wait_gather(b)
                compute(s, b)
                start_writeback(s, b)
            return carry

        lax.fori_loop(0, steps // NBUF, block, 0)
        wait_writeback(NBUF - 2)
        wait_writeback(NBUF - 1)

    return sc_gather


def kernel(service_indices, phases, amplitudes, service_emb,
           phase_W, phase_b, amp_W, amp_b, final_W, final_b):
    bn = service_indices.shape[0]
    idx = service_indices.astype(jnp.int32)
    ph = phases.reshape(-1)
    am = amplitudes.reshape(-1)
    pw = phase_W.reshape(1, Q)
    aw = amp_W.reshape(1, Q)
    pb = phase_b.reshape(1, Q)
    ab = amp_b.reshape(1, Q)
    fb = final_b.reshape(1, EMB)
    P, vp, va = _project_table(service_emb, final_W, pw, pb, aw, ab, fb)
    sc = _make_sc_gather(bn, service_emb.shape[0])
    return sc(P, idx, ph, am, vp, va)
